# baseline (speedup 1.0000x reference)
import functools

import jax
import jax.numpy as jnp
from jax import lax
from jax.experimental import pallas as pl
from jax.experimental.pallas import tpu as pltpu
from jax.experimental.pallas import tpu_sc as plsc

CAP = 100000
SEQ = 20
HID = 64
K = 32
L = 16
NC, NS = 2, 16
NW = NC * NS
TILE = 128
NT_FULL = CAP // TILE
REM = CAP - NT_FULL * TILE
CT = 5
COLS_PER_CHUNK = CT * TILE
NCHUNK = 5
BG = 10
GP_CHUNK = COLS_PER_CHUNK // L
MAXG_W = 25 * (TILE // L) + 2
BB = 10
NBMAX = (MAXG_W + BB - 1) // BB
NEG_INF = float("-inf")
BIG_I = 2**30


def _iota16():
    return lax.iota(jnp.int32, L)


def _splat_f(x):
    return jnp.full((L,), x, dtype=jnp.float32)


def _splat_i(x):
    return jnp.full((L,), x, dtype=jnp.int32)


def _rsqrt16(x):
    i = plsc.bitcast(x, jnp.int32)
    i = jnp.int32(0x5F3759DF) - (i >> 1)
    r = plsc.bitcast(i, jnp.float32)
    for _ in range(3):
        r = r * (1.5 - 0.5 * x * r * r)
    return r


_MESH = plsc.VectorSubcoreMesh(core_axis_name="c", subcore_axis_name="s")
_PARAMS = pltpu.CompilerParams(needs_layout_passes=False)


@functools.partial(
    pl.kernel,
    out_type=(
        jax.ShapeDtypeStruct((NW * K,), jnp.float32),
        jax.ShapeDtypeStruct((NW * K,), jnp.int32),
    ),
    mesh=_MESH,
    scratch_types=[
        pltpu.VMEM((HID, COLS_PER_CHUNK), jnp.float32),
        pltpu.VMEM((HID, COLS_PER_CHUNK), jnp.float32),
        pltpu.VMEM((HID, REM), jnp.float32),
        pltpu.VMEM((MAXG_W * L,), jnp.float32),
        pltpu.VMEM((HID * L,), jnp.float32),
        pltpu.VMEM((K,), jnp.float32),
        pltpu.VMEM((K,), jnp.int32),
        pltpu.VMEM((NBMAX * L,), jnp.float32),
        pltpu.SemaphoreType.DMA,
        pltpu.SemaphoreType.DMA,
    ],
    compiler_params=_PARAMS,
)
def _partial_topk(emb_hbm, tail_hbm, q_hbm, cval_hbm, cidx_hbm,
                  buf0, buf1, tbuf, sims, qv, cv, ci, bm, sem0, sem1):
    wid = lax.axis_index("s") * NC + lax.axis_index("c")
    t0 = (wid * NT_FULL) // NW
    n_t = ((wid + 1) * NT_FULL) // NW - t0
    iota = _iota16()

    pltpu.sync_copy(q_hbm, qv)

    def _dot_groups(src, col_base, sim_base, n):
        acc = [_splat_f(0.0) for _ in range(n)]
        nacc = [_splat_f(0.0) for _ in range(n)]
        for h in range(HID):
            qh = qv[pl.ds(h * L, L)]
            for s in range(n):
                v = src[h, pl.ds(col_base + s * L, L)]
                acc[s] = acc[s] + v * qh
                nacc[s] = nacc[s] + v * v
        for s in range(n):
            en = jnp.maximum(nacc[s] * _rsqrt16(nacc[s]), 1e-8)
            sims[pl.ds(sim_base + s * L, L)] = acc[s] / en

    def _loc_t(c):
        return jnp.minimum(c * CT, n_t - CT)

    def _start(c, dbuf, dsem):
        pltpu.async_copy(
            emb_hbm.at[:, pl.ds((t0 + _loc_t(c)) * TILE, COLS_PER_CHUNK)],
            dbuf, dsem,
        )

    def _drain(dbuf, dsem):
        pltpu.make_async_copy(
            emb_hbm.at[:, pl.ds(0, COLS_PER_CHUNK)], dbuf, dsem
        ).wait()

    def _compute(c, src):
        loc_t = _loc_t(c)

        def block_body(b, _):
            g0 = b * BG
            _dot_groups(src, g0 * L, (loc_t * (TILE // L) + g0) * L, BG)
            return 0

        lax.fori_loop(0, GP_CHUNK // BG, block_body, 0)

    _start(0, buf0, sem0)

    def chunk_body(c, _):
        @pl.when(c % 2 == 0)
        def _():
            _drain(buf0, sem0)

            @pl.when(c + 1 < NCHUNK)
            def _():
                _start(c + 1, buf1, sem1)

            _compute(c, buf0)

        @pl.when(c % 2 == 1)
        def _():
            _drain(buf1, sem1)

            @pl.when(c + 1 < NCHUNK)
            def _():
                _start(c + 1, buf0, sem0)

            _compute(c, buf1)

        return 0

    lax.fori_loop(0, NCHUNK, chunk_body, 0)

    n_g = n_t * (TILE // L)
    @pl.when(wid == NW - 1)
    def _():
        pltpu.sync_copy(tail_hbm, tbuf)
        _dot_groups(tbuf, 0, n_g * L, REM // L)

    n_gt = n_g + jnp.where(wid == NW - 1, REM // L, 0)

    mask0 = iota == 0
    base_elem = t0 * TILE
    n_b = (n_gt + BB - 1) // BB

    def _block_max(glo, ghi):
        def b_scan(g, m):
            return jnp.maximum(m, sims[pl.ds(g * L, L)])
        return lax.fori_loop(glo, ghi, b_scan, _splat_f(NEG_INF))

    def pass1(nb, _):
        bm[pl.ds(nb * L, L)] = _block_max(
            nb * BB, jnp.minimum(nb * BB + BB, n_gt)
        )
        return 0

    lax.fori_loop(0, n_b, pass1, 0)

    def select_body(j, _):
        def bm_scan(nb, mb):
            m, b = mb
            v = bm[pl.ds(nb * L, L)]
            upd = v > m
            return jnp.where(upd, v, m), jnp.where(upd, _splat_i(nb), b)

        m, b = lax.fori_loop(
            0, n_b, bm_scan, (_splat_f(NEG_INF), _splat_i(0))
        )
        mx = jnp.max(m)
        mxv = _splat_f(mx)
        bsel = jnp.min(jnp.where(m == mxv, b, BIG_I))
        glo = bsel * BB
        ghi = jnp.minimum(glo + BB, n_gt)

        def pos_scan(g, pv):
            v = sims[pl.ds(g * L, L)]
            idxv = _splat_i(g * L) + iota
            return jnp.minimum(pv, jnp.where(v == mxv, idxv, BIG_I))

        pos = jnp.min(lax.fori_loop(glo, ghi, pos_scan, _splat_i(2**30)))
        jv = _splat_i(j)
        plsc.store_scatter(cv, [jv], mxv, mask=mask0)
        plsc.store_scatter(ci, [jv], _splat_i(base_elem + pos), mask=mask0)
        plsc.store_scatter(sims, [_splat_i(pos)], _splat_f(NEG_INF), mask=mask0)
        bm[pl.ds(bsel * L, L)] = _block_max(glo, ghi)
        return 0

    lax.fori_loop(0, K, select_body, 0)

    pltpu.sync_copy(cv, cval_hbm.at[pl.ds(wid * K, K)])
    pltpu.sync_copy(ci, cidx_hbm.at[pl.ds(wid * K, K)])


def _merge_gather_body(cval_ref, cidx_ref, q_ref, epi_ref, tail_ref,
                       retr_ref, score_ref, *bufs_sems):
    cv = cval_ref[...].reshape(8, TILE)
    ci = cidx_ref[...].reshape(8, TILE)
    q = q_ref[...]
    qn = jnp.maximum(jnp.sqrt(jnp.sum(q * q)), 1e-8)

    nbuf = len(bufs_sems) // 2
    bufs, sems = bufs_sems[:nbuf], bufs_sems[nbuf:]

    def fetch(idx, s):
        t = jnp.minimum(idx // TILE, NT_FULL - 1)
        offs = pl.multiple_of(t * TILE, TILE)
        pltpu.make_async_copy(
            epi_ref.at[:, :, pl.ds(offs, TILE)], bufs[s], sems[s]
        ).start()

    slot = (lax.broadcasted_iota(jnp.int32, (8, TILE), 0) * TILE
            + lax.broadcasted_iota(jnp.int32, (8, TILE), 1))
    m = cv
    vals, idxs = [], []
    for jj in range(K):
        mx = jnp.max(m)
        pos = jnp.min(jnp.where(m == mx, slot, BIG_I))
        sel = slot == pos
        idxs.append(jnp.min(jnp.where(sel, ci, BIG_I)))
        vals.append(mx)
        m = jnp.where(sel, NEG_INF, m)
        if jj < nbuf:
            fetch(idxs[jj], jj)
    score_ref[...] = jnp.stack(vals) / qn

    lane_iota = lax.iota(jnp.int32, TILE)
    tail_iota = lax.iota(jnp.int32, REM)
    tail = tail_ref[...].reshape(SEQ * HID, REM)
    for j in range(K):
        s = j % nbuf
        pltpu.make_async_copy(
            epi_ref.at[:, :, pl.ds(0, TILE)], bufs[s], sems[s]
        ).wait()
        lane = idxs[j] - jnp.minimum(idxs[j] // TILE, NT_FULL - 1) * TILE
        onehot = (lane_iota == lane).astype(jnp.float32)
        col = bufs[s][...].reshape(SEQ * HID, TILE) @ onehot
        t_onehot = (tail_iota == (idxs[j] - NT_FULL * TILE)).astype(
            jnp.float32
        )
        col = col + tail @ t_onehot
        retr_ref[j] = col.reshape(SEQ, HID)
        if j + nbuf < K:
            fetch(idxs[j + nbuf], s)


def _merge_gather(cval, cidx, q0, epi_t, epi_tail):
    return pl.pallas_call(
        _merge_gather_body,
        in_specs=[
            pl.BlockSpec(memory_space=pltpu.VMEM),
            pl.BlockSpec(memory_space=pltpu.VMEM),
            pl.BlockSpec(memory_space=pltpu.VMEM),
            pl.BlockSpec(memory_space=pl.ANY),
            pl.BlockSpec(memory_space=pltpu.VMEM),
        ],
        out_specs=(
            pl.BlockSpec(memory_space=pltpu.VMEM),
            pl.BlockSpec(memory_space=pltpu.VMEM),
        ),
        out_shape=(
            jax.ShapeDtypeStruct((K, SEQ, HID), jnp.float32),
            jax.ShapeDtypeStruct((K,), jnp.float32),
        ),
        scratch_shapes=(
            [pltpu.VMEM((SEQ, HID, TILE), jnp.float32)] * 8
            + [pltpu.SemaphoreType.DMA] * 8
        ),
    )(cval, cidx, q0, epi_t, epi_tail)


def kernel(query, k, episodes, episode_embeddings):
    if query.ndim == 1:
        query = query[None, :]
    q0 = query[0]
    qb = jnp.repeat(q0, L)
    emb_t = episode_embeddings.T
    epi_t = jnp.transpose(episodes, (1, 2, 0))
    emb_tail = emb_t[:, NT_FULL * TILE:]
    epi_tail = epi_t[:, :, NT_FULL * TILE:]
    cval, cidx = _partial_topk(emb_t, emb_tail, qb)
    retr, scores = _merge_gather(cval, cidx, q0, epi_t, epi_tail)
    scores = scores + jnp.asarray(k - k, dtype=scores.dtype)
    return retr, scores
